# baseline (speedup 1.0000x reference)
import functools

import numpy as np
import jax
import jax.numpy as jnp
from jax import lax
from jax.experimental import pallas as pl
from jax.experimental.pallas import tpu as pltpu
from jax.experimental.pallas import tpu_sc as plsc

_N = 10000
_NP = 10240
_IN_C = 256
_HID = 256
_OUT = 128
_FW = 64
_DW = 16
_K = 10
_BN_EPS = 1e-5

_NC, _NS = 2, 16
_CH = 128
_CPT = 162
_EP = _NS * _CPT * _CH
_ROWS_PT = _NP // _NS
_RING = 5
_LAG = 3

_MESH = plsc.VectorSubcoreMesh(
    core_axis_name="c", subcore_axis_name="s", num_cores=_NC, num_subcores=_NS)
_MESH1 = plsc.VectorSubcoreMesh(
    core_axis_name="c", subcore_axis_name="s", num_cores=1, num_subcores=_NS)


def _fill_buf(buf, n_rows, width, value):
    vec = jnp.full((16,), value, jnp.float32)

    def body(i, _):
        for l in range(width // 16):
            buf[i, pl.ds(l * 16, 16)] = vec
        return 0

    lax.fori_loop(0, n_rows, body, 0)


def _zero_acc(zbuf, acc_sh, s):
    nfull, rem = _ROWS_PT // _CH, _ROWS_PT % _CH
    for r in range(nfull):
        pltpu.sync_copy(zbuf, acc_sh.at[pl.ds(s * _ROWS_PT + r * _CH, _CH)])
    if rem:
        pltpu.sync_copy(zbuf.at[pl.ds(0, rem)],
                        acc_sh.at[pl.ds(s * _ROWS_PT + nfull * _CH, rem)])


@functools.partial(
    pl.kernel,
    out_type=jax.ShapeDtypeStruct((_NP, _DW), jnp.float32),
    mesh=_MESH1,
    compiler_params=pltpu.CompilerParams(use_tc_tiling_on_sc=False),
    scratch_types=[
        pltpu.VMEM((_CPT, _CH), jnp.int32),
        pltpu.VMEM((2, _CH, _DW), jnp.float32),
        pltpu.VMEM_SHARED((_NP, _DW), jnp.float32),
        pltpu.SemaphoreType.DMA,
    ],
)
def _deg_kernel(dst_hbm, out_hbm, dst_v, zo, acc_sh, sem):
    s = lax.axis_index("s")
    pltpu.sync_copy(dst_hbm.at[s], dst_v)
    _fill_buf(zo.at[0], _CH, _DW, 0.0)
    _fill_buf(zo.at[1], _CH, _DW, 1.0)
    _zero_acc(zo.at[0], acc_sh, s)
    plsc.subcore_barrier()

    def start(j):
        pltpu.async_copy(zo.at[1], acc_sh.at[dst_v.at[j]], sem, add=True)

    def drain(j):
        pltpu.make_async_copy(zo.at[1], acc_sh.at[dst_v.at[j]], sem).wait()

    def group(gi, _):
        for q in range(8):
            start(gi * 8 + q)
        for q in range(8):
            drain(gi * 8 + q)
        return 0

    lax.fori_loop(0, _CPT // 8, group, 0)
    for j in range((_CPT // 8) * 8, _CPT):
        start(j)
        drain(j)
    plsc.subcore_barrier()
    pltpu.sync_copy(acc_sh.at[pl.ds(s * _ROWS_PT, _ROWS_PT)],
                    out_hbm.at[pl.ds(s * _ROWS_PT, _ROWS_PT)])


@functools.partial(
    pl.kernel,
    out_type=[jax.ShapeDtypeStruct((_NP, _FW), jnp.float32)] * 4,
    mesh=_MESH,
    compiler_params=pltpu.CompilerParams(use_tc_tiling_on_sc=False),
    scratch_types=[
        pltpu.VMEM((_CPT, _CH), jnp.int32),
        pltpu.VMEM((_CPT, _CH), jnp.int32),
        pltpu.VMEM((_RING, _CH, _FW), jnp.float32),
        pltpu.VMEM_SHARED((_NP, _FW), jnp.float32),
        pltpu.VMEM((16, 16), jnp.float32),
        pltpu.SemaphoreType.DMA((_RING,)),
        pltpu.SemaphoreType.DMA((_RING,)),
    ],
)
def _prop_kernel(g_a, g_b, src_hbm, dst_hbm, dinv_hbm, hid_a, hid_b, tvec,
                 ha_out, hb_out, ga_out, gb_out,
                 src_v, dst_v, rows, acc_sh, tsm, gs, ss):
    c = lax.axis_index("c")
    s = lax.axis_index("s")
    pltpu.sync_copy(src_hbm.at[s], src_v)
    pltpu.sync_copy(dst_hbm.at[s], dst_v)
    pltpu.sync_copy(tvec, tsm)
    _fill_buf(rows.at[0], _CH, _FW, 0.0)
    _zero_acc(rows.at[0], acc_sh, s)

    def pipeline(g_hbm):
        def g_start(j, b):
            pltpu.async_copy(g_hbm.at[src_v.at[j]], rows.at[b], gs.at[b])

        def g_wait(j, b):
            pltpu.make_async_copy(
                g_hbm.at[src_v.at[j]], rows.at[b], gs.at[b]).wait()

        def s_start(j, b):
            pltpu.async_copy(rows.at[b], acc_sh.at[dst_v.at[j]], ss.at[b],
                             add=True)

        def s_wait(j, b):
            pltpu.make_async_copy(
                rows.at[b], acc_sh.at[dst_v.at[j]], ss.at[b]).wait()

        la = _RING - _LAG

        def iteration(j, b):
            g_wait(j, b)
            s_start(j, b)
            nb = (b + la) % _RING
            if isinstance(j, int):
                if j >= _LAG:
                    s_wait(j - _LAG, nb)
                if j + la <= _CPT - 1:
                    g_start(j + la, nb)
            else:
                s_wait(j - _LAG, nb)

                @pl.when(j + la <= _CPT - 1)
                def _():
                    g_start(j + la, nb)

        for j in range(la):
            g_start(j, j)
        for j in range(_RING):
            iteration(j, j)

        def steady(m, _):
            for q in range(_RING):
                iteration(_RING * m + _RING + q, q)
            return 0

        nsteady = (_CPT - _RING) // _RING
        lax.fori_loop(0, nsteady, steady, 0)
        for j in range(_RING * nsteady + _RING, _CPT):
            iteration(j, j % _RING)
        for j in range(_CPT - _LAG, _CPT):
            s_wait(j, j % _RING)

    def combine(k, hid_ref, h_out, g_out):
        tkv = tsm[k, :]

        for r in range(_ROWS_PT // _CH):
            base = s * _ROWS_PT + r * _CH
            sl = pl.ds(base, _CH)
            pltpu.sync_copy(acc_sh.at[sl], rows.at[0])
            pltpu.sync_copy(dinv_hbm.at[sl], rows.at[1])
            pltpu.sync_copy(hid_ref.at[sl], rows.at[2])

            def body(i, _):
                for l in range(_FW // 16):
                    ls = pl.ds(l * 16, 16)
                    d = rows[1, i, ls]
                    h = rows[0, i, ls] * d
                    rows[2, i, ls] = rows[2, i, ls] + tkv * h
                    rows[0, i, ls] = d * h
                return 0

            lax.fori_loop(0, _CH, body, 0)
            pltpu.sync_copy(rows.at[2], h_out.at[sl])
            pltpu.sync_copy(rows.at[0], g_out.at[sl])
        _fill_buf(rows.at[0], _CH, _FW, 0.0)
        _zero_acc(rows.at[0], acc_sh, s)

    def round_step(k, g_src_a, g_src_b, hid_src_a, hid_src_b):
        plsc.subcore_barrier()

        @pl.when(c == 0)
        def _():
            pipeline(g_src_a)

        @pl.when(c == 1)
        def _():
            pipeline(g_src_b)

        plsc.subcore_barrier()

        @pl.when(c == 0)
        def _():
            combine(k, hid_src_a, ha_out, ga_out)

        @pl.when(c == 1)
        def _():
            combine(k, hid_src_b, hb_out, gb_out)

    round_step(1, g_a, g_b, hid_a, hid_b)

    def later_round(k, _):
        round_step(k, ga_out, gb_out, ha_out, hb_out)
        return 0

    lax.fori_loop(2, _K + 1, later_round, 0)


_BM = 1024


def _dinv_body(deg_ref, out_ref, out64_ref):
    i = pl.program_id(0)
    d = deg_ref[...][:, :1]
    row = lax.broadcasted_iota(jnp.int32, (_BM, 1), 0) + i * _BM
    dv = jnp.where((row < _N) & (d > 0), lax.rsqrt(d), 0.0)
    out_ref[...] = jnp.broadcast_to(dv, (_BM, _OUT))
    out64_ref[...] = jnp.broadcast_to(dv, (_BM, _FW))


def _dinv_call(deg2):
    return pl.pallas_call(
        _dinv_body,
        grid=(_NP // _BM,),
        in_specs=[pl.BlockSpec((_BM, _DW), lambda i: (i, 0))],
        out_specs=[pl.BlockSpec((_BM, _OUT), lambda i: (i, 0)),
                   pl.BlockSpec((_BM, _FW), lambda i: (i, 0))],
        out_shape=[jax.ShapeDtypeStruct((_NP, _OUT), jnp.float32),
                   jax.ShapeDtypeStruct((_NP, _FW), jnp.float32)],
    )(deg2)


def _mlp_body(x_ref, w1, b1, g1, be1, w2, b2, g2, be2, w3, b3,
              dinv_ref, t0, ga_out, gb_out, ha_out, hb_out):
    inv = np.float32(1.0 / np.sqrt(1.0 + _BN_EPS))
    h = jnp.dot(x_ref[...], w1[...], preferred_element_type=jnp.float32)
    h = jnp.maximum(h + b1[...], 0.0)
    h = h * (g1[...] * inv) + be1[...]
    h = jnp.dot(h, w2[...], preferred_element_type=jnp.float32)
    h = jnp.maximum(h + b2[...], 0.0)
    h = h * (g2[...] * inv) + be2[...]
    h = jnp.dot(h, w3[...], preferred_element_type=jnp.float32) + b3[...]
    hid = t0[0, 0] * h
    ha_out[...] = hid[:, :_FW]
    hb_out[...] = hid[:, _FW:]
    g = dinv_ref[...] * h
    ga_out[...] = g[:, :_FW]
    gb_out[...] = g[:, _FW:]


def _mlp_call(xp, W1, b1, g1, be1, W2, b2, g2, be2, W3, b3, dinv, t0):
    full = lambda shape: pl.BlockSpec(shape, lambda i, s=shape: tuple(0 for _ in s))
    return pl.pallas_call(
        _mlp_body,
        grid=(_NP // _BM,),
        in_specs=[
            pl.BlockSpec((_BM, _IN_C), lambda i: (i, 0)),
            full((_IN_C, _HID)), full((1, _HID)), full((1, _HID)), full((1, _HID)),
            full((_HID, _HID)), full((1, _HID)), full((1, _HID)), full((1, _HID)),
            full((_HID, _OUT)), full((1, _OUT)),
            pl.BlockSpec((_BM, _OUT), lambda i: (i, 0)),
            full((1, 1)),
        ],
        out_specs=[pl.BlockSpec((_BM, _FW), lambda i: (i, 0))] * 4,
        out_shape=[jax.ShapeDtypeStruct((_NP, _FW), jnp.float32)] * 4,
    )(xp, W1, b1, g1, be1, W2, b2, g2, be2, W3, b3, dinv, t0)


def kernel(x, edge_index, W1, b1, g1, be1, W2, b2, g2, be2, W3, b3, temp):
    src = edge_index[0].astype(jnp.int32)
    dst = edge_index[1].astype(jnp.int32)
    loop = jnp.arange(_N, dtype=jnp.int32)
    npad = _EP - _N - src.shape[0]
    pad = _N + (jnp.arange(npad, dtype=jnp.int32) % (_NP - _N))
    srcp = jnp.concatenate([src, loop, pad]).reshape(_NS, _CPT, _CH)
    dstp = jnp.concatenate([dst, loop, pad]).reshape(_NS, _CPT, _CH)
    xp = jnp.pad(x, ((0, _NP - _N), (0, 0)))
    tvec = jnp.broadcast_to(jnp.pad(temp, (0, 16 - temp.shape[0]))[:, None],
                            (16, 16))
    r = lambda v: v.reshape(1, -1)

    deg = _deg_kernel(dstp)
    dinv, dinv64 = _dinv_call(deg)
    g_a, g_b, hid_a, hid_b = _mlp_call(xp, W1, r(b1), r(g1), r(be1), W2,
                                       r(b2), r(g2), r(be2), W3, r(b3), dinv,
                                       temp[0:1].reshape(1, 1))
    hid_a, hid_b, _, _ = _prop_kernel(g_a, g_b, srcp, dstp, dinv64,
                                      hid_a, hid_b, tvec)
    return jnp.concatenate([hid_a, hid_b], axis=1)[:_N]


# speedup vs baseline: 1.2005x; 1.2005x over previous
import functools

import numpy as np
import jax
import jax.numpy as jnp
from jax import lax
from jax.experimental import pallas as pl
from jax.experimental.pallas import tpu as pltpu
from jax.experimental.pallas import tpu_sc as plsc

_N = 10000
_NP = 10240
_IN_C = 256
_HID = 256
_OUT = 128
_FW = 64
_DW = 16
_K = 10
_BN_EPS = 1e-5

_NC, _NS = 2, 16
_CH = 128
_CPT = 162
_EP = _NS * _CPT * _CH
_ROWS_PT = _NP // _NS
_RING = 5
_LAG = 1

_MESH = plsc.VectorSubcoreMesh(
    core_axis_name="c", subcore_axis_name="s", num_cores=_NC, num_subcores=_NS)
_MESH1 = plsc.VectorSubcoreMesh(
    core_axis_name="c", subcore_axis_name="s", num_cores=1, num_subcores=_NS)


def _fill_buf(buf, n_rows, width, value):
    vec = jnp.full((16,), value, jnp.float32)

    def body(i, _):
        for l in range(width // 16):
            buf[i, pl.ds(l * 16, 16)] = vec
        return 0

    lax.fori_loop(0, n_rows, body, 0)


def _zero_acc(zbuf, acc_sh, s):
    nfull, rem = _ROWS_PT // _CH, _ROWS_PT % _CH
    for r in range(nfull):
        pltpu.sync_copy(zbuf, acc_sh.at[pl.ds(s * _ROWS_PT + r * _CH, _CH)])
    if rem:
        pltpu.sync_copy(zbuf.at[pl.ds(0, rem)],
                        acc_sh.at[pl.ds(s * _ROWS_PT + nfull * _CH, rem)])


@functools.partial(
    pl.kernel,
    out_type=jax.ShapeDtypeStruct((_NP, _DW), jnp.float32),
    mesh=_MESH1,
    compiler_params=pltpu.CompilerParams(use_tc_tiling_on_sc=False),
    scratch_types=[
        pltpu.VMEM((_CPT, _CH), jnp.int32),
        pltpu.VMEM((2, _CH, _DW), jnp.float32),
        pltpu.VMEM_SHARED((_NP, _DW), jnp.float32),
        pltpu.SemaphoreType.DMA,
    ],
)
def _deg_kernel(dst_hbm, out_hbm, dst_v, zo, acc_sh, sem):
    s = lax.axis_index("s")
    pltpu.sync_copy(dst_hbm.at[s], dst_v)
    _fill_buf(zo.at[0], _CH, _DW, 0.0)
    _fill_buf(zo.at[1], _CH, _DW, 1.0)
    _zero_acc(zo.at[0], acc_sh, s)
    plsc.subcore_barrier()

    def start(j):
        pltpu.async_copy(zo.at[1], acc_sh.at[dst_v.at[j]], sem, add=True)

    def drain(j):
        pltpu.make_async_copy(zo.at[1], acc_sh.at[dst_v.at[j]], sem).wait()

    def group(gi, _):
        for q in range(8):
            start(gi * 8 + q)
        for q in range(8):
            drain(gi * 8 + q)
        return 0

    lax.fori_loop(0, _CPT // 8, group, 0)
    for j in range((_CPT // 8) * 8, _CPT):
        start(j)
        drain(j)
    plsc.subcore_barrier()
    pltpu.sync_copy(acc_sh.at[pl.ds(s * _ROWS_PT, _ROWS_PT)],
                    out_hbm.at[pl.ds(s * _ROWS_PT, _ROWS_PT)])


@functools.partial(
    pl.kernel,
    out_type=[jax.ShapeDtypeStruct((_NP, _FW), jnp.float32)] * 4,
    mesh=_MESH,
    compiler_params=pltpu.CompilerParams(use_tc_tiling_on_sc=False),
    scratch_types=[
        pltpu.VMEM((_CPT, _CH), jnp.int32),
        pltpu.VMEM((_CPT, _CH), jnp.int32),
        pltpu.VMEM((_RING, _CH, _FW), jnp.float32),
        pltpu.VMEM_SHARED((_NP, _FW), jnp.float32),
        pltpu.VMEM((16, 16), jnp.float32),
        pltpu.SemaphoreType.DMA((_RING,)),
        pltpu.SemaphoreType.DMA((_RING,)),
    ],
)
def _prop_kernel(g_a, g_b, src_hbm, dst_hbm, dinv_hbm, hid_a, hid_b, tvec,
                 ha_out, hb_out, ga_out, gb_out,
                 src_v, dst_v, rows, acc_sh, tsm, gs, ss):
    c = lax.axis_index("c")
    s = lax.axis_index("s")
    pltpu.sync_copy(src_hbm.at[s], src_v)
    pltpu.sync_copy(dst_hbm.at[s], dst_v)
    pltpu.sync_copy(tvec, tsm)
    _fill_buf(rows.at[0], _CH, _FW, 0.0)
    _zero_acc(rows.at[0], acc_sh, s)

    def pipeline(g_hbm):
        def g_start(j, b):
            pltpu.async_copy(g_hbm.at[src_v.at[j]], rows.at[b], gs.at[b])

        def g_wait(j, b):
            pltpu.make_async_copy(
                g_hbm.at[src_v.at[j]], rows.at[b], gs.at[b]).wait()

        def s_start(j, b):
            pltpu.async_copy(rows.at[b], acc_sh.at[dst_v.at[j]], ss.at[b],
                             add=True)

        def s_wait(j, b):
            pltpu.make_async_copy(
                rows.at[b], acc_sh.at[dst_v.at[j]], ss.at[b]).wait()

        la = _RING - _LAG

        def iteration(j, b):
            g_wait(j, b)
            s_start(j, b)
            nb = (b + la) % _RING
            if isinstance(j, int):
                if j >= _LAG:
                    s_wait(j - _LAG, nb)
                if j + la <= _CPT - 1:
                    g_start(j + la, nb)
            else:
                s_wait(j - _LAG, nb)

                @pl.when(j + la <= _CPT - 1)
                def _():
                    g_start(j + la, nb)

        for j in range(la):
            g_start(j, j)
        for j in range(_RING):
            iteration(j, j)

        def steady(m, _):
            for q in range(_RING):
                iteration(_RING * m + _RING + q, q)
            return 0

        nsteady = (_CPT - _RING) // _RING
        lax.fori_loop(0, nsteady, steady, 0)
        for j in range(_RING * nsteady + _RING, _CPT):
            iteration(j, j % _RING)
        for j in range(_CPT - _LAG, _CPT):
            s_wait(j, j % _RING)

    def combine(k, hid_ref, h_out, g_out):
        tkv = tsm[k, :]

        for r in range(_ROWS_PT // _CH):
            base = s * _ROWS_PT + r * _CH
            sl = pl.ds(base, _CH)
            pltpu.sync_copy(acc_sh.at[sl], rows.at[0])
            pltpu.sync_copy(dinv_hbm.at[sl], rows.at[1])
            pltpu.sync_copy(hid_ref.at[sl], rows.at[2])

            def body(i, _):
                for l in range(_FW // 16):
                    ls = pl.ds(l * 16, 16)
                    d = rows[1, i, ls]
                    h = rows[0, i, ls] * d
                    rows[2, i, ls] = rows[2, i, ls] + tkv * h
                    rows[0, i, ls] = d * h
                return 0

            lax.fori_loop(0, _CH, body, 0)
            pltpu.sync_copy(rows.at[2], h_out.at[sl])
            pltpu.sync_copy(rows.at[0], g_out.at[sl])
        _fill_buf(rows.at[0], _CH, _FW, 0.0)
        _zero_acc(rows.at[0], acc_sh, s)

    def round_step(k, g_src_a, g_src_b, hid_src_a, hid_src_b):
        plsc.subcore_barrier()

        @pl.when(c == 0)
        def _():
            pipeline(g_src_a)

        @pl.when(c == 1)
        def _():
            pipeline(g_src_b)

        plsc.subcore_barrier()

        @pl.when(c == 0)
        def _():
            combine(k, hid_src_a, ha_out, ga_out)

        @pl.when(c == 1)
        def _():
            combine(k, hid_src_b, hb_out, gb_out)

    round_step(1, g_a, g_b, hid_a, hid_b)

    def later_round(k, _):
        round_step(k, ga_out, gb_out, ha_out, hb_out)
        return 0

    lax.fori_loop(2, _K + 1, later_round, 0)


_BM = 1024


def _dinv_body(deg_ref, out_ref, out64_ref):
    i = pl.program_id(0)
    d = deg_ref[...][:, :1]
    row = lax.broadcasted_iota(jnp.int32, (_BM, 1), 0) + i * _BM
    dv = jnp.where((row < _N) & (d > 0), lax.rsqrt(d), 0.0)
    out_ref[...] = jnp.broadcast_to(dv, (_BM, _OUT))
    out64_ref[...] = jnp.broadcast_to(dv, (_BM, _FW))


def _dinv_call(deg2):
    return pl.pallas_call(
        _dinv_body,
        grid=(_NP // _BM,),
        in_specs=[pl.BlockSpec((_BM, _DW), lambda i: (i, 0))],
        out_specs=[pl.BlockSpec((_BM, _OUT), lambda i: (i, 0)),
                   pl.BlockSpec((_BM, _FW), lambda i: (i, 0))],
        out_shape=[jax.ShapeDtypeStruct((_NP, _OUT), jnp.float32),
                   jax.ShapeDtypeStruct((_NP, _FW), jnp.float32)],
    )(deg2)


def _mlp_body(x_ref, w1, b1, g1, be1, w2, b2, g2, be2, w3, b3,
              dinv_ref, t0, ga_out, gb_out, ha_out, hb_out):
    inv = np.float32(1.0 / np.sqrt(1.0 + _BN_EPS))
    h = jnp.dot(x_ref[...], w1[...], preferred_element_type=jnp.float32)
    h = jnp.maximum(h + b1[...], 0.0)
    h = h * (g1[...] * inv) + be1[...]
    h = jnp.dot(h, w2[...], preferred_element_type=jnp.float32)
    h = jnp.maximum(h + b2[...], 0.0)
    h = h * (g2[...] * inv) + be2[...]
    h = jnp.dot(h, w3[...], preferred_element_type=jnp.float32) + b3[...]
    hid = t0[0, 0] * h
    ha_out[...] = hid[:, :_FW]
    hb_out[...] = hid[:, _FW:]
    g = dinv_ref[...] * h
    ga_out[...] = g[:, :_FW]
    gb_out[...] = g[:, _FW:]


def _mlp_call(xp, W1, b1, g1, be1, W2, b2, g2, be2, W3, b3, dinv, t0):
    full = lambda shape: pl.BlockSpec(shape, lambda i, s=shape: tuple(0 for _ in s))
    return pl.pallas_call(
        _mlp_body,
        grid=(_NP // _BM,),
        in_specs=[
            pl.BlockSpec((_BM, _IN_C), lambda i: (i, 0)),
            full((_IN_C, _HID)), full((1, _HID)), full((1, _HID)), full((1, _HID)),
            full((_HID, _HID)), full((1, _HID)), full((1, _HID)), full((1, _HID)),
            full((_HID, _OUT)), full((1, _OUT)),
            pl.BlockSpec((_BM, _OUT), lambda i: (i, 0)),
            full((1, 1)),
        ],
        out_specs=[pl.BlockSpec((_BM, _FW), lambda i: (i, 0))] * 4,
        out_shape=[jax.ShapeDtypeStruct((_NP, _FW), jnp.float32)] * 4,
    )(xp, W1, b1, g1, be1, W2, b2, g2, be2, W3, b3, dinv, t0)


def kernel(x, edge_index, W1, b1, g1, be1, W2, b2, g2, be2, W3, b3, temp):
    src = edge_index[0].astype(jnp.int32)
    dst = edge_index[1].astype(jnp.int32)
    loop = jnp.arange(_N, dtype=jnp.int32)
    npad = _EP - _N - src.shape[0]
    pad = _N + (jnp.arange(npad, dtype=jnp.int32) % (_NP - _N))
    srcp = jnp.concatenate([src, loop, pad]).reshape(_NS, _CPT, _CH)
    dstp = jnp.concatenate([dst, loop, pad]).reshape(_NS, _CPT, _CH)
    xp = jnp.pad(x, ((0, _NP - _N), (0, 0)))
    tvec = jnp.broadcast_to(jnp.pad(temp, (0, 16 - temp.shape[0]))[:, None],
                            (16, 16))
    r = lambda v: v.reshape(1, -1)

    deg = _deg_kernel(dstp)
    dinv, dinv64 = _dinv_call(deg)
    g_a, g_b, hid_a, hid_b = _mlp_call(xp, W1, r(b1), r(g1), r(be1), W2,
                                       r(b2), r(g2), r(be2), W3, r(b3), dinv,
                                       temp[0:1].reshape(1, 1))
    hid_a, hid_b, _, _ = _prop_kernel(g_a, g_b, srcp, dstp, dinv64,
                                      hid_a, hid_b, tvec)
    return jnp.concatenate([hid_a, hid_b], axis=1)[:_N]


# speedup vs baseline: 1.2034x; 1.0024x over previous
import functools

import numpy as np
import jax
import jax.numpy as jnp
from jax import lax
from jax.experimental import pallas as pl
from jax.experimental.pallas import tpu as pltpu
from jax.experimental.pallas import tpu_sc as plsc

_N = 10000
_NP = 10240
_IN_C = 256
_HID = 256
_OUT = 128
_FW = 64
_DW = 16
_K = 10
_BN_EPS = 1e-5

_NC, _NS = 2, 16
_CH = 108
_CPT = 192
_EP = _NS * _CPT * _CH
_ROWS_PT = _NP // _NS
_RING = 6
_LAG = 1

_MESH = plsc.VectorSubcoreMesh(
    core_axis_name="c", subcore_axis_name="s", num_cores=_NC, num_subcores=_NS)
_MESH1 = plsc.VectorSubcoreMesh(
    core_axis_name="c", subcore_axis_name="s", num_cores=1, num_subcores=_NS)


def _fill_buf(buf, n_rows, width, value):
    vec = jnp.full((16,), value, jnp.float32)

    def body(i, _):
        for l in range(width // 16):
            buf[i, pl.ds(l * 16, 16)] = vec
        return 0

    lax.fori_loop(0, n_rows, body, 0)


def _zero_acc(zbuf, acc_sh, s):
    nfull, rem = _ROWS_PT // _CH, _ROWS_PT % _CH
    for r in range(nfull):
        pltpu.sync_copy(zbuf, acc_sh.at[pl.ds(s * _ROWS_PT + r * _CH, _CH)])
    if rem:
        pltpu.sync_copy(zbuf.at[pl.ds(0, rem)],
                        acc_sh.at[pl.ds(s * _ROWS_PT + nfull * _CH, rem)])


@functools.partial(
    pl.kernel,
    out_type=jax.ShapeDtypeStruct((_NP, _DW), jnp.float32),
    mesh=_MESH1,
    compiler_params=pltpu.CompilerParams(use_tc_tiling_on_sc=False),
    scratch_types=[
        pltpu.VMEM((_CPT, _CH), jnp.int32),
        pltpu.VMEM((2, _CH, _DW), jnp.float32),
        pltpu.VMEM_SHARED((_NP, _DW), jnp.float32),
        pltpu.SemaphoreType.DMA,
    ],
)
def _deg_kernel(dst_hbm, out_hbm, dst_v, zo, acc_sh, sem):
    s = lax.axis_index("s")
    pltpu.sync_copy(dst_hbm.at[s], dst_v)
    _fill_buf(zo.at[0], _CH, _DW, 0.0)
    _fill_buf(zo.at[1], _CH, _DW, 1.0)
    _zero_acc(zo.at[0], acc_sh, s)
    plsc.subcore_barrier()

    def start(j):
        pltpu.async_copy(zo.at[1], acc_sh.at[dst_v.at[j]], sem, add=True)

    def drain(j):
        pltpu.make_async_copy(zo.at[1], acc_sh.at[dst_v.at[j]], sem).wait()

    def group(gi, _):
        for q in range(8):
            start(gi * 8 + q)
        for q in range(8):
            drain(gi * 8 + q)
        return 0

    lax.fori_loop(0, _CPT // 8, group, 0)
    for j in range((_CPT // 8) * 8, _CPT):
        start(j)
        drain(j)
    plsc.subcore_barrier()
    pltpu.sync_copy(acc_sh.at[pl.ds(s * _ROWS_PT, _ROWS_PT)],
                    out_hbm.at[pl.ds(s * _ROWS_PT, _ROWS_PT)])


@functools.partial(
    pl.kernel,
    out_type=[jax.ShapeDtypeStruct((_NP, _FW), jnp.float32)] * 4,
    mesh=_MESH,
    compiler_params=pltpu.CompilerParams(use_tc_tiling_on_sc=False),
    scratch_types=[
        pltpu.VMEM((_CPT, _CH), jnp.int32),
        pltpu.VMEM((_CPT, _CH), jnp.int32),
        pltpu.VMEM((_RING, _CH, _FW), jnp.float32),
        pltpu.VMEM_SHARED((_NP, _FW), jnp.float32),
        pltpu.VMEM((16, 16), jnp.float32),
        pltpu.SemaphoreType.DMA((_RING,)),
        pltpu.SemaphoreType.DMA((_RING,)),
    ],
)
def _prop_kernel(g_a, g_b, src_hbm, dst_hbm, dinv_hbm, hid_a, hid_b, tvec,
                 ha_out, hb_out, ga_out, gb_out,
                 src_v, dst_v, rows, acc_sh, tsm, gs, ss):
    c = lax.axis_index("c")
    s = lax.axis_index("s")
    pltpu.sync_copy(src_hbm.at[s], src_v)
    pltpu.sync_copy(dst_hbm.at[s], dst_v)
    pltpu.sync_copy(tvec, tsm)
    _fill_buf(rows.at[0], _CH, _FW, 0.0)
    _zero_acc(rows.at[0], acc_sh, s)

    def pipeline(g_hbm):
        def g_start(j, b):
            pltpu.async_copy(g_hbm.at[src_v.at[j]], rows.at[b], gs.at[b])

        def g_wait(j, b):
            pltpu.make_async_copy(
                g_hbm.at[src_v.at[j]], rows.at[b], gs.at[b]).wait()

        def s_start(j, b):
            pltpu.async_copy(rows.at[b], acc_sh.at[dst_v.at[j]], ss.at[b],
                             add=True)

        def s_wait(j, b):
            pltpu.make_async_copy(
                rows.at[b], acc_sh.at[dst_v.at[j]], ss.at[b]).wait()

        la = _RING - _LAG

        def iteration(j, b):
            g_wait(j, b)
            s_start(j, b)
            nb = (b + la) % _RING
            if isinstance(j, int):
                if j >= _LAG:
                    s_wait(j - _LAG, nb)
                if j + la <= _CPT - 1:
                    g_start(j + la, nb)
            else:
                s_wait(j - _LAG, nb)

                @pl.when(j + la <= _CPT - 1)
                def _():
                    g_start(j + la, nb)

        for j in range(la):
            g_start(j, j)
        for j in range(_RING):
            iteration(j, j)

        def steady(m, _):
            for q in range(_RING):
                iteration(_RING * m + _RING + q, q)
            return 0

        nsteady = (_CPT - _RING) // _RING
        lax.fori_loop(0, nsteady, steady, 0)
        for j in range(_RING * nsteady + _RING, _CPT):
            iteration(j, j % _RING)
        for j in range(_CPT - _LAG, _CPT):
            s_wait(j, j % _RING)

    def combine(k, hid_ref, h_out, g_out):
        tkv = tsm[k, :]

        nfull, rem = _ROWS_PT // _CH, _ROWS_PT % _CH
        chunks = [(r * _CH, _CH) for r in range(nfull)]
        if rem:
            chunks.append((nfull * _CH, rem))
        for off, nr in chunks:
            base = s * _ROWS_PT + off
            sl = pl.ds(base, nr)
            bsl = pl.ds(0, nr)
            pltpu.sync_copy(acc_sh.at[sl], rows.at[0].at[bsl])
            pltpu.sync_copy(dinv_hbm.at[sl], rows.at[1].at[bsl])
            pltpu.sync_copy(hid_ref.at[sl], rows.at[2].at[bsl])

            def body(i, _):
                for l in range(_FW // 16):
                    ls = pl.ds(l * 16, 16)
                    d = rows[1, i, ls]
                    h = rows[0, i, ls] * d
                    rows[2, i, ls] = rows[2, i, ls] + tkv * h
                    rows[0, i, ls] = d * h
                return 0

            lax.fori_loop(0, nr, body, 0)
            pltpu.sync_copy(rows.at[2].at[bsl], h_out.at[sl])
            pltpu.sync_copy(rows.at[0].at[bsl], g_out.at[sl])
        _fill_buf(rows.at[0], _CH, _FW, 0.0)
        _zero_acc(rows.at[0], acc_sh, s)

    def round_step(k, g_src_a, g_src_b, hid_src_a, hid_src_b):
        plsc.subcore_barrier()

        @pl.when(c == 0)
        def _():
            pipeline(g_src_a)

        @pl.when(c == 1)
        def _():
            pipeline(g_src_b)

        plsc.subcore_barrier()

        @pl.when(c == 0)
        def _():
            combine(k, hid_src_a, ha_out, ga_out)

        @pl.when(c == 1)
        def _():
            combine(k, hid_src_b, hb_out, gb_out)

    round_step(1, g_a, g_b, hid_a, hid_b)

    def later_round(k, _):
        round_step(k, ga_out, gb_out, ha_out, hb_out)
        return 0

    lax.fori_loop(2, _K + 1, later_round, 0)


_BM = 1024


def _dinv_body(deg_ref, out_ref, out64_ref):
    i = pl.program_id(0)
    d = deg_ref[...][:, :1]
    row = lax.broadcasted_iota(jnp.int32, (_BM, 1), 0) + i * _BM
    dv = jnp.where((row < _N) & (d > 0), lax.rsqrt(d), 0.0)
    out_ref[...] = jnp.broadcast_to(dv, (_BM, _OUT))
    out64_ref[...] = jnp.broadcast_to(dv, (_BM, _FW))


def _dinv_call(deg2):
    return pl.pallas_call(
        _dinv_body,
        grid=(_NP // _BM,),
        in_specs=[pl.BlockSpec((_BM, _DW), lambda i: (i, 0))],
        out_specs=[pl.BlockSpec((_BM, _OUT), lambda i: (i, 0)),
                   pl.BlockSpec((_BM, _FW), lambda i: (i, 0))],
        out_shape=[jax.ShapeDtypeStruct((_NP, _OUT), jnp.float32),
                   jax.ShapeDtypeStruct((_NP, _FW), jnp.float32)],
    )(deg2)


def _mlp_body(x_ref, w1, b1, g1, be1, w2, b2, g2, be2, w3, b3,
              dinv_ref, t0, ga_out, gb_out, ha_out, hb_out):
    inv = np.float32(1.0 / np.sqrt(1.0 + _BN_EPS))
    h = jnp.dot(x_ref[...], w1[...], preferred_element_type=jnp.float32)
    h = jnp.maximum(h + b1[...], 0.0)
    h = h * (g1[...] * inv) + be1[...]
    h = jnp.dot(h, w2[...], preferred_element_type=jnp.float32)
    h = jnp.maximum(h + b2[...], 0.0)
    h = h * (g2[...] * inv) + be2[...]
    h = jnp.dot(h, w3[...], preferred_element_type=jnp.float32) + b3[...]
    hid = t0[0, 0] * h
    ha_out[...] = hid[:, :_FW]
    hb_out[...] = hid[:, _FW:]
    g = dinv_ref[...] * h
    ga_out[...] = g[:, :_FW]
    gb_out[...] = g[:, _FW:]


def _mlp_call(xp, W1, b1, g1, be1, W2, b2, g2, be2, W3, b3, dinv, t0):
    full = lambda shape: pl.BlockSpec(shape, lambda i, s=shape: tuple(0 for _ in s))
    return pl.pallas_call(
        _mlp_body,
        grid=(_NP // _BM,),
        in_specs=[
            pl.BlockSpec((_BM, _IN_C), lambda i: (i, 0)),
            full((_IN_C, _HID)), full((1, _HID)), full((1, _HID)), full((1, _HID)),
            full((_HID, _HID)), full((1, _HID)), full((1, _HID)), full((1, _HID)),
            full((_HID, _OUT)), full((1, _OUT)),
            pl.BlockSpec((_BM, _OUT), lambda i: (i, 0)),
            full((1, 1)),
        ],
        out_specs=[pl.BlockSpec((_BM, _FW), lambda i: (i, 0))] * 4,
        out_shape=[jax.ShapeDtypeStruct((_NP, _FW), jnp.float32)] * 4,
    )(xp, W1, b1, g1, be1, W2, b2, g2, be2, W3, b3, dinv, t0)


def kernel(x, edge_index, W1, b1, g1, be1, W2, b2, g2, be2, W3, b3, temp):
    src = edge_index[0].astype(jnp.int32)
    dst = edge_index[1].astype(jnp.int32)
    loop = jnp.arange(_N, dtype=jnp.int32)
    npad = _EP - _N - src.shape[0]
    pad = _N + (jnp.arange(npad, dtype=jnp.int32) % (_NP - _N))
    srcp = jnp.concatenate([src, loop, pad]).reshape(_NS, _CPT, _CH)
    dstp = jnp.concatenate([dst, loop, pad]).reshape(_NS, _CPT, _CH)
    xp = jnp.pad(x, ((0, _NP - _N), (0, 0)))
    tvec = jnp.broadcast_to(jnp.pad(temp, (0, 16 - temp.shape[0]))[:, None],
                            (16, 16))
    r = lambda v: v.reshape(1, -1)

    deg = _deg_kernel(dstp)
    dinv, dinv64 = _dinv_call(deg)
    g_a, g_b, hid_a, hid_b = _mlp_call(xp, W1, r(b1), r(g1), r(be1), W2,
                                       r(b2), r(g2), r(be2), W3, r(b3), dinv,
                                       temp[0:1].reshape(1, 1))
    hid_a, hid_b, _, _ = _prop_kernel(g_a, g_b, srcp, dstp, dinv64,
                                      hid_a, hid_b, tvec)
    return jnp.concatenate([hid_a, hid_b], axis=1)[:_N]
